# baseline (speedup 1.0000x reference)
import functools

import jax
import jax.numpy as jnp
from jax import lax
from jax.experimental import pallas as pl
from jax.experimental.pallas import tpu as pltpu
from jax.experimental.pallas import tpu_sc as plsc

N_VOCAB = 1000000
N_CONTEXT = 2048
N_STATE = 64
BATCH = 8

NUM_CORES = 2
NUM_SUBCORES = 16
NUM_WORKERS = NUM_CORES * NUM_SUBCORES
POS_PER_W = N_CONTEXT // NUM_WORKERS
LANES = 16
COLS = N_STATE // LANES

_mesh = plsc.VectorSubcoreMesh(core_axis_name="c", subcore_axis_name="s")


@functools.partial(
    pl.kernel,
    out_type=jax.ShapeDtypeStruct((BATCH * N_CONTEXT, N_STATE), jnp.float32),
    mesh=_mesh,
    compiler_params=pltpu.CompilerParams(use_tc_tiling_on_sc=False),
    scratch_types=[
        pltpu.VMEM((BATCH, POS_PER_W), jnp.int32),
        pltpu.VMEM((POS_PER_W, N_STATE), jnp.float32),
        pltpu.VMEM((BATCH * POS_PER_W, N_STATE), jnp.float32),
        pltpu.SemaphoreType.DMA,
        pltpu.SemaphoreType.DMA,
        pltpu.SemaphoreType.DMA,
    ],
)
def _embed(x_hbm, voc_hbm, pos_hbm, out_hbm, idx_v, pos_v, rows_v, sem_i, sem_g, sem_o):
    w = lax.axis_index("s") * NUM_CORES + lax.axis_index("c")
    p0 = w * POS_PER_W

    idx_cps = [
        pltpu.async_copy(x_hbm.at[b, pl.ds(p0, POS_PER_W)], idx_v.at[b], sem_i)
        for b in range(BATCH)
    ]
    pos_cp = pltpu.async_copy(pos_hbm.at[pl.ds(p0, POS_PER_W)], pos_v, sem_o)

    for cp in idx_cps:
        cp.wait()
    gather_cps = [
        pltpu.async_copy(
            voc_hbm.at[idx_v.at[b]],
            rows_v.at[pl.ds(b * POS_PER_W, POS_PER_W)],
            sem_g,
        )
        for b in range(BATCH)
    ]
    pos_cp.wait()
    for cp in gather_cps:
        cp.wait()

    def add_body(k, _):
        for j in range(COLS):
            sl = pl.ds(j * LANES, LANES)
            p = pos_v[k, sl]
            for b in range(BATCH):
                r = b * POS_PER_W + k
                rows_v[r, sl] = rows_v[r, sl] + p
        return 0

    lax.fori_loop(0, POS_PER_W, add_body, 0)

    out_cps = [
        pltpu.async_copy(
            rows_v.at[pl.ds(b * POS_PER_W, POS_PER_W)],
            out_hbm.at[pl.ds(b * N_CONTEXT + p0, POS_PER_W)],
            sem_o,
        )
        for b in range(BATCH)
    ]
    for cp in out_cps:
        cp.wait()


def kernel(x, w_voc, w_pos):
    out = _embed(x, w_voc, w_pos)
    return out.reshape(BATCH, N_CONTEXT, N_STATE)


# speedup vs baseline: 1.0028x; 1.0028x over previous
import functools

import jax
import jax.numpy as jnp
from jax import lax
from jax.experimental import layout as jex_layout
from jax.experimental import pallas as pl
from jax.experimental.pallas import tpu as pltpu
from jax.experimental.pallas import tpu_sc as plsc

N_VOCAB = 1000000
N_CONTEXT = 2048
N_STATE = 64
BATCH = 8

NUM_CORES = 2
NUM_SUBCORES = 16
NUM_WORKERS = NUM_CORES * NUM_SUBCORES
POS_PER_W = N_CONTEXT // NUM_WORKERS
LANES = 16
COLS = N_STATE // LANES

_mesh = plsc.VectorSubcoreMesh(core_axis_name="c", subcore_axis_name="s")


@functools.partial(
    pl.kernel,
    out_type=jax.ShapeDtypeStruct((BATCH * N_CONTEXT, N_STATE), jnp.float32),
    mesh=_mesh,
    compiler_params=pltpu.CompilerParams(use_tc_tiling_on_sc=False),
    scratch_types=[
        pltpu.VMEM((BATCH, POS_PER_W), jnp.int32),
        pltpu.VMEM((POS_PER_W, N_STATE), jnp.float32),
        pltpu.VMEM((BATCH * POS_PER_W, N_STATE), jnp.float32),
        pltpu.SemaphoreType.DMA,
        pltpu.SemaphoreType.DMA,
        pltpu.SemaphoreType.DMA,
    ],
)
def _embed(x_hbm, voc_hbm, pos_hbm, out_hbm, idx_v, pos_v, rows_v, sem_i, sem_g, sem_o):
    w = lax.axis_index("s") * NUM_CORES + lax.axis_index("c")
    p0 = w * POS_PER_W

    idx_cps = [
        pltpu.async_copy(x_hbm.at[b, pl.ds(p0, POS_PER_W)], idx_v.at[b], sem_i)
        for b in range(BATCH)
    ]
    pos_cp = pltpu.async_copy(pos_hbm.at[pl.ds(p0, POS_PER_W)], pos_v, sem_o)

    for cp in idx_cps:
        cp.wait()
    gather_cps = [
        pltpu.async_copy(
            voc_hbm.at[idx_v.at[b]],
            rows_v.at[pl.ds(b * POS_PER_W, POS_PER_W)],
            sem_g,
        )
        for b in range(BATCH)
    ]
    pos_cp.wait()
    for cp in gather_cps:
        cp.wait()

    def add_body(k, _):
        for j in range(COLS):
            sl = pl.ds(j * LANES, LANES)
            p = pos_v[k, sl]
            for b in range(BATCH):
                r = b * POS_PER_W + k
                rows_v[r, sl] = rows_v[r, sl] + p
        return 0

    lax.fori_loop(0, POS_PER_W, add_body, 0)

    out_cps = [
        pltpu.async_copy(
            rows_v.at[pl.ds(b * POS_PER_W, POS_PER_W)],
            out_hbm.at[pl.ds(b * N_CONTEXT + p0, POS_PER_W)],
            sem_o,
        )
        for b in range(BATCH)
    ]
    for cp in out_cps:
        cp.wait()


def kernel(x, w_voc, w_pos):
    voc_sc = jax.device_put(
        w_voc,
        jex_layout.Format(
            jex_layout.Layout(major_to_minor=(0, 1), tiling=((8,),)),
            jax.sharding.SingleDeviceSharding(jax.local_devices()[0]),
        ),
    )
    out = _embed(x, voc_sc, w_pos)
    return out.reshape(BATCH, N_CONTEXT, N_STATE)


# speedup vs baseline: 3.0658x; 3.0572x over previous
import functools

import jax
import jax.numpy as jnp
from jax import lax
from jax.experimental import pallas as pl
from jax.experimental.pallas import tpu as pltpu
from jax.experimental.pallas import tpu_sc as plsc

N_VOCAB = 1000000
N_CONTEXT = 2048
N_STATE = 64
BATCH = 8

NUM_CORES = 2
NUM_SUBCORES = 16
NUM_WORKERS = NUM_CORES * NUM_SUBCORES

PBLK = 128
N_PBLK = N_CONTEXT // PBLK
N_UNITS = BATCH * N_PBLK
UNITS_PER_W = N_UNITS // NUM_WORKERS
NBUF = 8
GROUPS = PBLK // 16
FCHUNKS = N_STATE // 16

_mesh = plsc.VectorSubcoreMesh(core_axis_name="c", subcore_axis_name="s")


@functools.partial(
    pl.kernel,
    out_type=jax.ShapeDtypeStruct((BATCH, N_STATE, N_CONTEXT), jnp.float32),
    mesh=_mesh,
    compiler_params=pltpu.CompilerParams(
        use_tc_tiling_on_sc=True, needs_layout_passes=False
    ),
    scratch_types=[
        pltpu.VMEM((PBLK,), jnp.int32),
        pltpu.VMEM((NBUF, N_STATE, 128), jnp.float32),
        pltpu.VMEM((N_STATE, PBLK), jnp.float32),
        pltpu.SemaphoreType.DMA,
        pltpu.SemaphoreType.DMA,
    ],
)
def _embed(x_hbm, vocT_hbm, posT_hbm, outT_hbm, ids_v, col_v, blk_v, sem_c, sem_m):
    w = lax.axis_index("s") * NUM_CORES + lax.axis_index("c")
    iota = lax.iota(jnp.int32, 16)

    def unit_body(u, carry):
        b = u // N_PBLK
        t0 = (u % N_PBLK) * PBLK

        pltpu.async_copy(x_hbm.at[b, pl.ds(t0, PBLK)], ids_v, sem_m).wait()
        pos_cp = pltpu.async_copy(posT_hbm.at[:, pl.ds(t0, PBLK)], blk_v, sem_m)

        offs = []
        lanes = []
        for g in range(GROUPS):
            idv = ids_v[pl.ds(g * 16, 16)]
            off16 = idv & jnp.int32(-128)
            offs.append(off16)
            lanes.append(idv - off16)

        cps = [None] * PBLK

        def issue(s):
            off = pl.multiple_of(offs[s // 16][s % 16], 128)
            cps[s] = pltpu.async_copy(
                vocT_hbm.at[:, pl.ds(off, 128)], col_v.at[s % NBUF], sem_c
            )

        for s in range(NBUF):
            issue(s)
        pos_cp.wait()

        for s in range(PBLK):
            cps[s].wait()
            lvec = jnp.full((16,), lanes[s // 16][s % 16], jnp.int32)
            bvec = jnp.full((16,), s % NBUF, jnp.int32)
            svec = jnp.full((16,), s, jnp.int32)
            for f in range(FCHUNKS):
                rows = iota + jnp.int32(f * 16)
                vals = plsc.load_gather(col_v, [bvec, rows, lvec])
                plsc.addupdate_scatter(blk_v, [rows, svec], vals)
            if s + NBUF < PBLK:
                issue(s + NBUF)

        pltpu.async_copy(blk_v, outT_hbm.at[b, :, pl.ds(t0, PBLK)], sem_m).wait()
        return carry

    lax.fori_loop(w * UNITS_PER_W, (w + 1) * UNITS_PER_W, unit_body, 0)


def kernel(x, w_voc, w_pos):
    out_t = _embed(x, w_voc.T, w_pos.T)
    return out_t.transpose(0, 2, 1)
